# baseline (speedup 1.0000x reference)
import functools

import jax
import jax.numpy as jnp
from jax import lax
from jax.experimental import pallas as pl
from jax.experimental.pallas import tpu as pltpu
from jax.experimental.pallas import tpu_sc as plsc

_NC = 2
_NS = 16
_NW = _NC * _NS

_K = 5
_SLOT = _K * 128


def _emb_grid(n_rows, d_model):
    rows_per_w = n_rows // _NW
    n_steps = rows_per_w // _K
    n_super = n_steps // 2
    slot_elems = _SLOT * d_model
    mesh = plsc.VectorSubcoreMesh(core_axis_name="c", subcore_axis_name="s")

    @functools.partial(
        pl.kernel,
        mesh=mesh,
        out_type=jax.ShapeDtypeStruct((n_rows * 128, d_model), jnp.float32),
        scratch_types=[
            pltpu.VMEM((rows_per_w, 128), jnp.int32),
            pltpu.VMEM((2, _SLOT, d_model), jnp.float32),
            pltpu.SemaphoreType.DMA,
            pltpu.SemaphoreType.DMA,
            pltpu.SemaphoreType.DMA,
            pltpu.SemaphoreType.DMA,
        ],
        compiler_params=pltpu.CompilerParams(use_tc_tiling_on_sc=False),
    )
    def emb(idx_hbm, table_hbm, out_hbm, idx_v, rows_v, g0, g1, s0, s1):
        wid = lax.axis_index("s") * _NC + lax.axis_index("c")
        base_row = wid * rows_per_w
        base_tok = base_row * 128
        gsem = (g0, g1)
        ssem = (s0, s1)

        pltpu.sync_copy(idx_hbm.at[pl.ds(base_row, rows_per_w)], idx_v)

        def fire_gathers(local_row, slot):
            for j in range(_K):
                pltpu.async_copy(
                    table_hbm.at[idx_v.at[local_row + j]],
                    rows_v.at[slot, pl.ds(j * 128, 128)],
                    gsem[slot],
                )

        def wait_gathers(slot):
            pltpu.make_async_copy(
                out_hbm.at[pl.ds(0, _SLOT)], rows_v.at[slot], gsem[slot]
            ).wait()

        def fire_store(tok0, slot):
            pltpu.async_copy(
                rows_v.at[slot], out_hbm.at[pl.ds(tok0, _SLOT)], ssem[slot]
            )

        def wait_store(slot):
            pltpu.make_async_copy(
                rows_v.at[slot], out_hbm.at[pl.ds(0, _SLOT)], ssem[slot]
            ).wait()

        fire_gathers(0, 0)
        fire_gathers(_K, 1)
        wait_gathers(0)
        fire_store(base_tok, 0)

        def superstep(t, carry):
            for b in range(2):
                k = 2 * t + b
                wait_store(b)
                fire_gathers(k * _K, b)
                wait_gathers(1 - b)
                fire_store(base_tok + (k - 1) * _SLOT, 1 - b)
            return carry

        lax.fori_loop(1, n_super, superstep, 0)

        wait_gathers(1)
        fire_store(base_tok + (n_steps - 1) * _SLOT, 1)
        wait_store(0)
        wait_store(1)

    return emb


def kernel(tokens, token_emb):
    b, s = tokens.shape
    v, d = token_emb.shape
    n = b * s
    n_rows = n // 128
    idx = tokens.reshape(n_rows, 128).astype(jnp.int32)
    out = _emb_grid(n_rows, d)(idx, token_emb)
    return out.reshape(b, s, d)
